# Initial kernel scaffold; baseline (speedup 1.0000x reference)
import jax
import jax.numpy as jnp
from jax.experimental import pallas as pl


def kernel(x_hidden, edge_index, W_l, b_l, W_r, W_mlp, b_mlp):
    raise NotImplementedError("write your pallas kernel here")


# speedup vs baseline: 2.3765x; 2.3765x over previous
import functools

import jax
import jax.numpy as jnp
from jax import lax
from jax.experimental import pallas as pl
from jax.experimental.pallas import tpu as pltpu
from jax.experimental.pallas import tpu_sc as plsc

N = 10000
D = 256
HALF = 128
E = 160000
NT = 16
CHUNK = 64
EP = 10112
NCHUNKS = EP // CHUNK
EPAD = EP * NT
R = 10112
RPT = R // NT
RCHUNKS = tuple((o, min(64, 632 - o)) for o in range(0, 632, 64))

_f32 = jnp.float32
_i32 = jnp.int32


def _sc_aggregate(pk, x2):
  mesh = plsc.VectorSubcoreMesh(core_axis_name="c", subcore_axis_name="s")

  @functools.partial(
      pl.kernel,
      out_type=[
          jax.ShapeDtypeStruct((2 * R, HALF), _f32),
      ],
      mesh=mesh,
      scratch_types=[
          pltpu.VMEM((CHUNK,), _i32),
          pltpu.VMEM((CHUNK,), _i32),
          pltpu.VMEM((CHUNK,), _i32),
          pltpu.VMEM((CHUNK, HALF), _f32),
          pltpu.VMEM_SHARED((R, HALF), _f32),
      ],
  )
  def agg(pk_hbm, x_hbm, out_hbm,
          pkv, srcv, dstv, rows, acc):
    c = lax.axis_index("c")
    s = lax.axis_index("s")
    zero16 = jnp.zeros((16,), _f32)
    one16 = jnp.ones((16,), _f32)

    def fill_rows(i, _):
      for jj in range(HALF // 16):
        rows[i, pl.ds(jj * 16, 16)] = zero16
      return 0
    lax.fori_loop(0, CHUNK, fill_rows, 0)

    for off, sz in RCHUNKS:
      pltpu.sync_copy(rows.at[pl.ds(0, sz)],
                      acc.at[pl.ds(s * RPT + off, sz)])

    plsc.subcore_barrier()

    coff = c * N

    def body(j, _):
      pltpu.sync_copy(pk_hbm.at[s, j], pkv)
      for k in range(CHUNK // 16):
        p = pkv[pl.ds(k * 16, 16)]
        srcv[pl.ds(k * 16, 16)] = lax.shift_right_logical(p, 14) + coff
        dstv[pl.ds(k * 16, 16)] = lax.bitwise_and(p, 16383)
      pltpu.sync_copy(x_hbm.at[srcv], rows)
      pltpu.sync_copy(rows, acc.at[dstv], add=True)
      return 0
    lax.fori_loop(0, NCHUNKS, body, 0)

    plsc.subcore_barrier()

    for off, sz in RCHUNKS:
      pltpu.sync_copy(acc.at[pl.ds(s * RPT + off, sz)], rows.at[pl.ds(0, sz)])
      pltpu.sync_copy(rows.at[pl.ds(0, sz)],
                      out_hbm.at[pl.ds(c * R + s * RPT + off, sz)])

  return agg(pk, x2)


def _tc_mlp(sums, cnt, x, wlt, wrt, wmt, bl, bm):
  BR = 632
  grid = (R // BR,)

  def gelu(h):
    return 0.5 * h * (1.0 + lax.erf(h * 0.7071067811865476))

  def body(sl_ref, sr_ref, cnt_ref, x_ref, wlt_ref, wrt_ref, wmt_ref,
           bl_ref, bm_ref, out_ref):
    inv = 1.0 / jnp.maximum(cnt_ref[:, 0:1], 1.0)
    ml = sl_ref[...] * inv
    mr = sr_ref[...] * inv
    h = jnp.dot(ml, wlt_ref[0:HALF, :], preferred_element_type=_f32)
    h = h + jnp.dot(mr, wlt_ref[HALF:D, :], preferred_element_type=_f32)
    h = h + jnp.dot(x_ref[...], wrt_ref[...], preferred_element_type=_f32)
    h = h + bl_ref[...]
    h = gelu(h)
    h = jnp.dot(h, wmt_ref[...], preferred_element_type=_f32) + bm_ref[...]
    out_ref[...] = gelu(h)

  return pl.pallas_call(
      body,
      grid=grid,
      in_specs=[
          pl.BlockSpec((BR, HALF), lambda i: (i, 0)),
          pl.BlockSpec((BR, HALF), lambda i: (i + R // BR, 0)),
          pl.BlockSpec((BR, 16), lambda i: (i, 0)),
          pl.BlockSpec((BR, D), lambda i: (i, 0)),
          pl.BlockSpec((D, D), lambda i: (0, 0)),
          pl.BlockSpec((D, D), lambda i: (0, 0)),
          pl.BlockSpec((D, D), lambda i: (0, 0)),
          pl.BlockSpec((1, D), lambda i: (0, 0)),
          pl.BlockSpec((1, D), lambda i: (0, 0)),
      ],
      out_specs=pl.BlockSpec((BR, D), lambda i: (i, 0)),
      out_shape=jax.ShapeDtypeStruct((N, D), _f32),
  )(sums, sums, cnt, x, wlt, wrt, wmt, bl, bm)


def kernel(x_hidden, edge_index, W_l, b_l, W_r, W_mlp, b_mlp):
  src = edge_index[0].astype(_i32)
  dst = edge_index[1].astype(_i32)
  pad = EPAD - E
  packed = src * 16384 + dst
  pk = jnp.concatenate([packed, jnp.full((pad,), N, _i32)]).reshape(
      NT, NCHUNKS, CHUNK)
  x2 = jnp.concatenate([x_hidden[:, :HALF], x_hidden[:, HALF:]], axis=0)
  (sums,) = _sc_aggregate(pk, x2)
  ones_e = jnp.ones((E,), _f32)
  cnt1 = jax.ops.segment_sum(ones_e, dst, num_segments=N)
  cnt = jnp.broadcast_to(jnp.pad(cnt1, (0, R - N))[:, None], (R, 16))
  return _tc_mlp(sums, cnt, x_hidden, W_l.T, W_r.T, W_mlp.T,
                 b_l.reshape(1, D), b_mlp.reshape(1, D))


# speedup vs baseline: 2.9011x; 1.2207x over previous
import functools

import jax
import jax.numpy as jnp
from jax import lax
from jax.experimental import pallas as pl
from jax.experimental.pallas import tpu as pltpu
from jax.experimental.pallas import tpu_sc as plsc

N = 10000
D = 256
HALF = 128
E = 160000
NT = 16
CHUNK = 64
EP = 10112
NCHUNKS = EP // CHUNK
EPAD = EP * NT
RMAIN = 10112
CBASE = RMAIN
R = RMAIN + 128
RPT = R // NT
RCHUNKS = tuple((o, 64) for o in range(0, RPT, 64))

_f32 = jnp.float32
_i32 = jnp.int32


def _sc_aggregate(pk, x2):
  mesh = plsc.VectorSubcoreMesh(core_axis_name="c", subcore_axis_name="s")

  @functools.partial(
      pl.kernel,
      out_type=[
          jax.ShapeDtypeStruct((2 * R, HALF), _f32),
      ],
      mesh=mesh,
      compiler_params=pltpu.CompilerParams(needs_layout_passes=False),
      scratch_types=[
          pltpu.VMEM((CHUNK,), _i32),
          pltpu.VMEM((CHUNK,), _i32),
          pltpu.VMEM((CHUNK,), _i32),
          pltpu.VMEM((CHUNK,), _i32),
          pltpu.VMEM((CHUNK, HALF), _f32),
          pltpu.VMEM((CHUNK, HALF), _f32),
          pltpu.VMEM_SHARED((R, HALF), _f32),
      ],
  )
  def agg(pk_hbm, x_hbm, out_hbm, pkv, srcv, dstv, crowv, rows, onehot, acc):
    c = lax.axis_index("c")
    s = lax.axis_index("s")
    zero16 = jnp.zeros((16,), _f32)
    one16 = jnp.ones((16,), _f32)
    iot = lax.iota(_i32, 16)

    def fill_rows(i, _):
      for jj in range(HALF // 16):
        rows[i, pl.ds(jj * 16, 16)] = zero16
        onehot[i, pl.ds(jj * 16, 16)] = zero16
      return 0
    lax.fori_loop(0, CHUNK, fill_rows, 0)

    for off, sz in RCHUNKS:
      pltpu.sync_copy(rows.at[pl.ds(0, sz)],
                      acc.at[pl.ds(s * RPT + off, sz)])

    plsc.subcore_barrier()

    coff = c * N

    def body(j, _):
      pltpu.sync_copy(pk_hbm.at[s, j], pkv)
      for k in range(CHUNK // 16):
        p = pkv[pl.ds(k * 16, 16)]
        dv = lax.bitwise_and(p, 16383)
        srcv[pl.ds(k * 16, 16)] = lax.shift_right_logical(p, 14) + coff
        dstv[pl.ds(k * 16, 16)] = dv
        crowv[pl.ds(k * 16, 16)] = lax.bitwise_and(dv, 127) + CBASE
        plsc.store_scatter(onehot,
                           [iot + (k * 16), lax.shift_right_logical(dv, 7)],
                           one16)
      pltpu.sync_copy(x_hbm.at[srcv], rows)
      pltpu.sync_copy(rows, acc.at[dstv], add=True)
      pltpu.sync_copy(onehot, acc.at[crowv], add=True)
      for k in range(CHUNK // 16):
        dv = dstv[pl.ds(k * 16, 16)]
        plsc.store_scatter(onehot,
                           [iot + (k * 16), lax.shift_right_logical(dv, 7)],
                           zero16)
      return 0
    lax.fori_loop(0, NCHUNKS, body, 0)

    plsc.subcore_barrier()

    for off, sz in RCHUNKS:
      pltpu.sync_copy(acc.at[pl.ds(s * RPT + off, sz)], rows.at[pl.ds(0, sz)])
      pltpu.sync_copy(rows.at[pl.ds(0, sz)],
                      out_hbm.at[pl.ds(c * R + s * RPT + off, sz)])

  return agg(pk, x2)


def _tc_mlp(sums, x, wlt, wrt, wmt, bl, bm):
  BR = 128
  grid = (79,)

  def gelu(h):
    return 0.5 * h * (1.0 + lax.erf(h * 0.7071067811865476))

  def body(sl_ref, sr_ref, cnt_ref, x_ref, wlt_ref, wrt_ref, wmt_ref,
           bl_ref, bm_ref, out_ref):
    pid = pl.program_id(0)
    sel = (lax.broadcasted_iota(_i32, (HALF, 1), 0) == pid).astype(_f32)
    cnt = jnp.dot(cnt_ref[...], sel, preferred_element_type=_f32)
    inv = 1.0 / jnp.maximum(cnt, 1.0)
    ml = sl_ref[...] * inv
    mr = sr_ref[...] * inv
    h = jnp.dot(ml, wlt_ref[0:HALF, :], preferred_element_type=_f32)
    h = h + jnp.dot(mr, wlt_ref[HALF:D, :], preferred_element_type=_f32)
    h = h + jnp.dot(x_ref[...], wrt_ref[...], preferred_element_type=_f32)
    h = h + bl_ref[...]
    h = gelu(h)
    h = jnp.dot(h, wmt_ref[...], preferred_element_type=_f32) + bm_ref[...]
    out_ref[...] = gelu(h)

  return pl.pallas_call(
      body,
      grid=grid,
      in_specs=[
          pl.BlockSpec((BR, HALF), lambda i: (i, 0)),
          pl.BlockSpec((BR, HALF), lambda i: (i + R // BR, 0)),
          pl.BlockSpec((BR, HALF), lambda i: (CBASE // BR, 0)),
          pl.BlockSpec((BR, D), lambda i: (i, 0)),
          pl.BlockSpec((D, D), lambda i: (0, 0)),
          pl.BlockSpec((D, D), lambda i: (0, 0)),
          pl.BlockSpec((D, D), lambda i: (0, 0)),
          pl.BlockSpec((1, D), lambda i: (0, 0)),
          pl.BlockSpec((1, D), lambda i: (0, 0)),
      ],
      out_specs=pl.BlockSpec((BR, D), lambda i: (i, 0)),
      out_shape=jax.ShapeDtypeStruct((N, D), _f32),
  )(sums, sums, sums, x, wlt, wrt, wmt, bl, bm)


def kernel(x_hidden, edge_index, W_l, b_l, W_r, W_mlp, b_mlp):
  src = edge_index[0].astype(_i32)
  dst = edge_index[1].astype(_i32)
  pad = EPAD - E
  packed = src * 16384 + dst
  pk = jnp.concatenate([packed, jnp.full((pad,), N, _i32)]).reshape(
      NT, NCHUNKS, CHUNK)
  x2 = jnp.concatenate([x_hidden[:, :HALF], x_hidden[:, HALF:]], axis=0)
  (sums,) = _sc_aggregate(pk, x2)
  return _tc_mlp(sums, x_hidden, W_l.T, W_r.T, W_mlp.T,
                 b_l.reshape(1, D), b_mlp.reshape(1, D))


# speedup vs baseline: 3.2603x; 1.1238x over previous
import functools

import jax
import jax.numpy as jnp
from jax import lax
from jax.experimental import pallas as pl
from jax.experimental.pallas import tpu as pltpu
from jax.experimental.pallas import tpu_sc as plsc

N = 10000
D = 256
HALF = 128
E = 160000
NT = 16
CHUNK = 64
EP = 10112
NCHUNKS = EP // CHUNK
EPAD = EP * NT
RMAIN = 10112
CBASE = RMAIN
R = RMAIN + 128
RPT = R // NT
RCHUNKS = tuple((o, 64) for o in range(0, RPT, 64))

_f32 = jnp.float32
_i32 = jnp.int32


def _sc_aggregate(pk, x2):
  mesh = plsc.VectorSubcoreMesh(core_axis_name="c", subcore_axis_name="s")

  @functools.partial(
      pl.kernel,
      out_type=[
          jax.ShapeDtypeStruct((2 * R, HALF), _f32),
      ],
      mesh=mesh,
      compiler_params=pltpu.CompilerParams(needs_layout_passes=False),
      scratch_types=[
          pltpu.VMEM((CHUNK,), _i32),
          pltpu.VMEM((CHUNK,), _i32),
          pltpu.VMEM((CHUNK,), _i32),
          pltpu.VMEM((HALF,), _i32),
          pltpu.VMEM((CHUNK, HALF), _f32),
          pltpu.VMEM((HALF, HALF), _f32),
          pltpu.VMEM_SHARED((R, HALF), _f32),
      ],
  )
  def agg(pk_hbm, x_hbm, out_hbm, pkv, srcv, dstv, crowv, rows, hist, acc):
    c = lax.axis_index("c")
    s = lax.axis_index("s")
    zero16 = jnp.zeros((16,), _f32)
    one16 = jnp.ones((16,), _f32)
    iot = lax.iota(_i32, 16)

    def fill_hist(i, _):
      for jj in range(HALF // 16):
        hist[i, pl.ds(jj * 16, 16)] = zero16
      return 0
    lax.fori_loop(0, HALF, fill_hist, 0)

    def fill_rows(i, _):
      for jj in range(HALF // 16):
        rows[i, pl.ds(jj * 16, 16)] = zero16
      return 0
    lax.fori_loop(0, CHUNK, fill_rows, 0)

    for k in range(HALF // 16):
      crowv[pl.ds(k * 16, 16)] = iot + (k * 16 + CBASE)

    for off, sz in RCHUNKS:
      pltpu.sync_copy(rows.at[pl.ds(0, sz)],
                      acc.at[pl.ds(s * RPT + off, sz)])

    plsc.subcore_barrier()

    coff = c * N

    def body(j, _):
      pltpu.sync_copy(pk_hbm.at[s, j], pkv)
      for k in range(CHUNK // 16):
        p = pkv[pl.ds(k * 16, 16)]
        dv = lax.bitwise_and(p, 16383)
        srcv[pl.ds(k * 16, 16)] = lax.shift_right_logical(p, 14) + coff
        dstv[pl.ds(k * 16, 16)] = dv
        cnts, lmask = plsc.scan_count(dv)
        plsc.addupdate_scatter(
            hist,
            [lax.bitwise_and(dv, 127), lax.shift_right_logical(dv, 7)],
            cnts.astype(_f32), mask=lmask)
      pltpu.sync_copy(x_hbm.at[srcv], rows)
      pltpu.sync_copy(rows, acc.at[dstv], add=True)
      return 0
    lax.fori_loop(0, NCHUNKS, body, 0)

    pltpu.sync_copy(hist, acc.at[crowv], add=True)

    plsc.subcore_barrier()

    for off, sz in RCHUNKS:
      pltpu.sync_copy(acc.at[pl.ds(s * RPT + off, sz)], rows.at[pl.ds(0, sz)])
      pltpu.sync_copy(rows.at[pl.ds(0, sz)],
                      out_hbm.at[pl.ds(c * R + s * RPT + off, sz)])

  return agg(pk, x2)


def _tc_mlp(sums, x, wlt, wrt, wmt, bl, bm):
  BR = 128
  grid = (79,)

  def gelu(h):
    return 0.5 * h * (1.0 + lax.erf(h * 0.7071067811865476))

  def body(sl_ref, sr_ref, cnt_ref, x_ref, wlt_ref, wrt_ref, wmt_ref,
           bl_ref, bm_ref, out_ref):
    pid = pl.program_id(0)
    sel = (lax.broadcasted_iota(_i32, (HALF, 1), 0) == pid).astype(_f32)
    cnt = jnp.dot(cnt_ref[...], sel, preferred_element_type=_f32)
    inv = 1.0 / jnp.maximum(cnt, 1.0)
    ml = sl_ref[...] * inv
    mr = sr_ref[...] * inv
    h = jnp.dot(ml, wlt_ref[0:HALF, :], preferred_element_type=_f32)
    h = h + jnp.dot(mr, wlt_ref[HALF:D, :], preferred_element_type=_f32)
    h = h + jnp.dot(x_ref[...], wrt_ref[...], preferred_element_type=_f32)
    h = h + bl_ref[...]
    h = gelu(h)
    h = jnp.dot(h, wmt_ref[...], preferred_element_type=_f32) + bm_ref[...]
    out_ref[...] = gelu(h)

  return pl.pallas_call(
      body,
      grid=grid,
      in_specs=[
          pl.BlockSpec((BR, HALF), lambda i: (i, 0)),
          pl.BlockSpec((BR, HALF), lambda i: (i + R // BR, 0)),
          pl.BlockSpec((BR, HALF), lambda i: (CBASE // BR, 0)),
          pl.BlockSpec((BR, D), lambda i: (i, 0)),
          pl.BlockSpec((D, D), lambda i: (0, 0)),
          pl.BlockSpec((D, D), lambda i: (0, 0)),
          pl.BlockSpec((D, D), lambda i: (0, 0)),
          pl.BlockSpec((1, D), lambda i: (0, 0)),
          pl.BlockSpec((1, D), lambda i: (0, 0)),
      ],
      out_specs=pl.BlockSpec((BR, D), lambda i: (i, 0)),
      out_shape=jax.ShapeDtypeStruct((N, D), _f32),
  )(sums, sums, sums, x, wlt, wrt, wmt, bl, bm)


def kernel(x_hidden, edge_index, W_l, b_l, W_r, W_mlp, b_mlp):
  src = edge_index[0].astype(_i32)
  dst = edge_index[1].astype(_i32)
  pad = EPAD - E
  packed = src * 16384 + dst
  pk = jnp.concatenate([packed, jnp.full((pad,), N, _i32)]).reshape(
      NT, NCHUNKS, CHUNK)
  x2 = jnp.concatenate([x_hidden[:, :HALF], x_hidden[:, HALF:]], axis=0)
  (sums,) = _sc_aggregate(pk, x2)
  return _tc_mlp(sums, x_hidden, W_l.T, W_r.T, W_mlp.T,
                 b_l.reshape(1, D), b_mlp.reshape(1, D))


# speedup vs baseline: 3.7806x; 1.1596x over previous
import functools

import jax
import jax.numpy as jnp
from jax import lax
from jax.experimental import pallas as pl
from jax.experimental.pallas import tpu as pltpu
from jax.experimental.pallas import tpu_sc as plsc

N = 10000
D = 256
HALF = 128
E = 160000
NT = 16
CHUNK = 32
EP = 10112
NCHUNKS = EP // CHUNK
EPAD = EP * NT
RMAIN = 10112
CBASE = RMAIN
R = RMAIN + 128
RPT = R // NT
RCHUNKS = tuple((o, 32) for o in range(0, RPT, 32))

_f32 = jnp.float32
_i32 = jnp.int32


def _sc_aggregate(pk, x2):
  mesh = plsc.VectorSubcoreMesh(core_axis_name="c", subcore_axis_name="s")

  @functools.partial(
      pl.kernel,
      out_type=[
          jax.ShapeDtypeStruct((2 * R, HALF), _f32),
      ],
      mesh=mesh,
      compiler_params=pltpu.CompilerParams(needs_layout_passes=False),
      scratch_types=[
          pltpu.VMEM((CHUNK,), _i32),
          pltpu.VMEM((CHUNK,), _i32),
          pltpu.VMEM((CHUNK,), _i32),
          pltpu.VMEM((CHUNK,), _i32),
          pltpu.VMEM((CHUNK,), _i32),
          pltpu.VMEM((HALF,), _i32),
          pltpu.VMEM((CHUNK, HALF), _f32),
          pltpu.VMEM((CHUNK, HALF), _f32),
          pltpu.VMEM((HALF, HALF), _f32),
          pltpu.VMEM_SHARED((R, HALF), _f32),
          pltpu.SemaphoreType.DMA,
          pltpu.SemaphoreType.DMA,
      ],
  )
  def agg(pk_hbm, x_hbm, out_hbm, pkv, srcv0, srcv1, dstv0, dstv1, crowv,
          rows0, rows1, hist, acc, gsem0, gsem1):
    srcv = (srcv0, srcv1)
    dstv = (dstv0, dstv1)
    rows = (rows0, rows1)
    gsem = (gsem0, gsem1)
    c = lax.axis_index("c")
    s = lax.axis_index("s")
    zero16 = jnp.zeros((16,), _f32)
    one16 = jnp.ones((16,), _f32)
    iot = lax.iota(_i32, 16)

    def fill_hist(i, _):
      for jj in range(HALF // 16):
        hist[i, pl.ds(jj * 16, 16)] = zero16
      return 0
    lax.fori_loop(0, HALF, fill_hist, 0)

    def fill_rows(i, _):
      for jj in range(HALF // 16):
        rows0[i, pl.ds(jj * 16, 16)] = zero16
      return 0
    lax.fori_loop(0, CHUNK, fill_rows, 0)

    for k in range(HALF // 16):
      crowv[pl.ds(k * 16, 16)] = iot + (k * 16 + CBASE)

    for off, sz in RCHUNKS:
      pltpu.sync_copy(rows0.at[pl.ds(0, sz)],
                      acc.at[pl.ds(s * RPT + off, sz)])

    plsc.subcore_barrier()

    coff = c * N

    def prep(i, b):
      pltpu.sync_copy(pk_hbm.at[s, i], pkv)
      for k in range(CHUNK // 16):
        p = pkv[pl.ds(k * 16, 16)]
        dv = lax.bitwise_and(p, 16383)
        srcv[b][pl.ds(k * 16, 16)] = lax.shift_right_logical(p, 14) + coff
        dstv[b][pl.ds(k * 16, 16)] = dv
        cnts, lmask = plsc.scan_count(dv)
        plsc.addupdate_scatter(
            hist,
            [lax.bitwise_and(dv, 127), lax.shift_right_logical(dv, 7)],
            cnts.astype(_f32), mask=lmask)

    prep(0, 0)
    pltpu.async_copy(x_hbm.at[srcv[0]], rows[0], gsem[0])
    prep(1, 1)
    pltpu.async_copy(x_hbm.at[srcv[1]], rows[1], gsem[1])

    def pair(t, _):
      for b in range(2):
        i = 2 * t + b
        pltpu.make_async_copy(x_hbm.at[srcv[b]], rows[b], gsem[b]).wait()
        pltpu.sync_copy(rows[b], acc.at[dstv[b]], add=True)

        @pl.when(i + 2 < NCHUNKS)
        def _():
          prep(i + 2, b)
          pltpu.async_copy(x_hbm.at[srcv[b]], rows[b], gsem[b])
      return 0
    lax.fori_loop(0, NCHUNKS // 2, pair, 0)

    pltpu.sync_copy(hist, acc.at[crowv], add=True)

    plsc.subcore_barrier()

    for off, sz in RCHUNKS:
      pltpu.sync_copy(acc.at[pl.ds(s * RPT + off, sz)], rows0.at[pl.ds(0, sz)])
      pltpu.sync_copy(rows0.at[pl.ds(0, sz)],
                      out_hbm.at[pl.ds(c * R + s * RPT + off, sz)])

  return agg(pk, x2)


def _tc_mlp(sums, x, wlt, wrt, wmt, bl, bm):
  BR = 128
  grid = (79,)

  def gelu(h):
    return 0.5 * h * (1.0 + lax.erf(h * 0.7071067811865476))

  def body(sl_ref, sr_ref, cnt_ref, x_ref, wlt_ref, wrt_ref, wmt_ref,
           bl_ref, bm_ref, out_ref):
    pid = pl.program_id(0)
    sel = (lax.broadcasted_iota(_i32, (HALF, 1), 0) == pid).astype(_f32)
    cnt = jnp.dot(cnt_ref[...], sel, preferred_element_type=_f32)
    inv = 1.0 / jnp.maximum(cnt, 1.0)
    ml = sl_ref[...] * inv
    mr = sr_ref[...] * inv
    h = jnp.dot(ml, wlt_ref[0:HALF, :], preferred_element_type=_f32)
    h = h + jnp.dot(mr, wlt_ref[HALF:D, :], preferred_element_type=_f32)
    h = h + jnp.dot(x_ref[...], wrt_ref[...], preferred_element_type=_f32)
    h = h + bl_ref[...]
    h = gelu(h)
    h = jnp.dot(h, wmt_ref[...], preferred_element_type=_f32) + bm_ref[...]
    out_ref[...] = gelu(h)

  return pl.pallas_call(
      body,
      grid=grid,
      in_specs=[
          pl.BlockSpec((BR, HALF), lambda i: (i, 0)),
          pl.BlockSpec((BR, HALF), lambda i: (i + R // BR, 0)),
          pl.BlockSpec((BR, HALF), lambda i: (CBASE // BR, 0)),
          pl.BlockSpec((BR, D), lambda i: (i, 0)),
          pl.BlockSpec((D, D), lambda i: (0, 0)),
          pl.BlockSpec((D, D), lambda i: (0, 0)),
          pl.BlockSpec((D, D), lambda i: (0, 0)),
          pl.BlockSpec((1, D), lambda i: (0, 0)),
          pl.BlockSpec((1, D), lambda i: (0, 0)),
      ],
      out_specs=pl.BlockSpec((BR, D), lambda i: (i, 0)),
      out_shape=jax.ShapeDtypeStruct((N, D), _f32),
  )(sums, sums, sums, x, wlt, wrt, wmt, bl, bm)


def kernel(x_hidden, edge_index, W_l, b_l, W_r, W_mlp, b_mlp):
  src = edge_index[0].astype(_i32)
  dst = edge_index[1].astype(_i32)
  pad = EPAD - E
  packed = src * 16384 + dst
  pk = jnp.concatenate([packed, jnp.full((pad,), N, _i32)]).reshape(
      NT, NCHUNKS, CHUNK)
  x2 = jnp.concatenate([x_hidden[:, :HALF], x_hidden[:, HALF:]], axis=0)
  (sums,) = _sc_aggregate(pk, x2)
  return _tc_mlp(sums, x_hidden, W_l.T, W_r.T, W_mlp.T,
                 b_l.reshape(1, D), b_mlp.reshape(1, D))
